# baseline (speedup 1.0000x reference)
import functools

import jax
import jax.numpy as jnp
from jax import lax
from jax.experimental import pallas as pl
from jax.experimental.pallas import tpu as pltpu
from jax.experimental.pallas import tpu_sc as plsc

_RETENTION = 0.9
_SPEED = 5.0
_BS = 2048
_SC_CHUNK = 64


def _attn_body(x_ref, m_ref, out_ref, acc_ref, sumw_ref, nsteps):
    i = pl.program_id(0)
    m = m_ref[...]
    x = x_ref[...]

    num = lax.dot_general(x, m, (((1,), (1,)), ((), ())),
                          preferred_element_type=jnp.float32)
    x_norm = jnp.sqrt(jnp.sum(x * x, axis=1, keepdims=True))
    m_norm = jnp.sqrt(jnp.sum(m * m, axis=1)).reshape(1, -1)
    denom = jnp.maximum(x_norm * m_norm, 1e-8)
    w = jnp.exp(_SPEED * (num / denom))

    part = lax.dot_general(w, m, (((1,), (0,)), ((), ())),
                           preferred_element_type=jnp.float32)
    wsum = jnp.sum(w, axis=1, keepdims=True)

    @pl.when(i == 0)
    def _init():
        acc_ref[...] = part
        sumw_ref[...] = wsum

    @pl.when(i > 0)
    def _accum():
        acc_ref[...] += part
        sumw_ref[...] += wsum

    @pl.when(i == nsteps - 1)
    def _final():
        out_ref[...] = acc_ref[...] / sumw_ref[...]


def _attention(x, memory_bank):
    size, dim = memory_bank.shape
    b = x.shape[0]
    bs = _BS if size % _BS == 0 else size
    nsteps = size // bs
    return pl.pallas_call(
        functools.partial(_attn_body, nsteps=nsteps),
        grid=(nsteps,),
        in_specs=[
            pl.BlockSpec((b, dim), lambda i: (0, 0)),
            pl.BlockSpec((bs, dim), lambda i: (i, 0)),
        ],
        out_specs=pl.BlockSpec((b, dim), lambda i: (0, 0)),
        out_shape=jax.ShapeDtypeStruct((b, dim), jnp.float32),
        scratch_shapes=[
            pltpu.VMEM((b, dim), jnp.float32),
            pltpu.VMEM((b, 1), jnp.float32),
        ],
    )(x, memory_bank)


def _sc_update(x_new, importance, memory_bank):
    size, dim = memory_bank.shape
    info = plsc.get_sparse_core_info()
    ns = info.num_subcores
    nw = info.num_cores * ns
    rows_per_w = size // nw
    chunk = _SC_CHUNK
    nchunks = rows_per_w // chunk
    thresh = 1.0 - _RETENTION
    mesh = plsc.VectorSubcoreMesh(core_axis_name="c", subcore_axis_name="s")

    @functools.partial(
        pl.kernel, mesh=mesh,
        out_type=jax.ShapeDtypeStruct((size, dim), jnp.float32),
        scratch_types=[
            pltpu.VMEM((rows_per_w,), jnp.float32),
            pltpu.MemorySpace.VMEM_SHARED((ns, 2, chunk, dim), jnp.float32),
            pltpu.SemaphoreType.DMA,
            pltpu.SemaphoreType.DMA,
            pltpu.SemaphoreType.DMA,
            pltpu.SemaphoreType.DMA,
        ],
    )
    def upd(xnew_hbm, imp_hbm, bank_hbm, out_hbm, imp_v, spm,
            in_a, in_b, out_a, out_b):
        sid = lax.axis_index("s")
        wid = sid * info.num_cores + lax.axis_index("c")
        w_base = wid * rows_per_w
        pltpu.sync_copy(imp_hbm.at[pl.ds(w_base, rows_per_w)], imp_v)
        buf_a = spm.at[sid, 0]
        buf_b = spm.at[sid, 1]

        def fill(c, buf, sem):
            base = w_base + c * chunk
            for g in range(chunk // 16):
                impv = imp_v[pl.ds(c * chunk + g * 16, 16)]
                for k in range(16):
                    r = g * 16 + k
                    s = impv[k]

                    @pl.when(s > thresh)
                    def _from_new():
                        pltpu.make_async_copy(
                            xnew_hbm.at[base + r], buf.at[r], sem).start()

                    @pl.when(jnp.logical_not(s > thresh))
                    def _from_bank():
                        pltpu.make_async_copy(
                            bank_hbm.at[base + r], buf.at[r], sem).start()

        def drain(buf, sem):
            pltpu.make_async_copy(
                bank_hbm.at[pl.ds(0, chunk), :], buf, sem).wait()

        def flush(c, buf, sem):
            pltpu.make_async_copy(
                buf, out_hbm.at[pl.ds(w_base + c * chunk, chunk), :],
                sem).start()

        def wait_flush(buf, sem):
            pltpu.make_async_copy(
                bank_hbm.at[pl.ds(0, chunk), :], buf, sem).wait()

        def do_pair(i, _):
            c0 = 2 * i
            c1 = 2 * i + 1

            @pl.when(i > 0)
            def _reuse_a():
                wait_flush(buf_a, out_a)
            fill(c0, buf_a, in_a)

            @pl.when(i > 0)
            def _reuse_b():
                wait_flush(buf_b, out_b)
            fill(c1, buf_b, in_b)

            drain(buf_a, in_a)
            flush(c0, buf_a, out_a)
            drain(buf_b, in_b)
            flush(c1, buf_b, out_b)
            return 0

        lax.fori_loop(0, nchunks // 2, do_pair, 0)
        wait_flush(buf_a, out_a)
        wait_flush(buf_b, out_b)

    return upd(x_new, importance, memory_bank)


def kernel(x, x_new, importance, memory_bank):
    out = _attention(x, memory_bank)
    new_bank = _sc_update(x_new, importance, memory_bank)
    return out, new_bank


# speedup vs baseline: 1.2179x; 1.2179x over previous
import functools

import jax
import jax.numpy as jnp
from jax import lax
from jax.experimental import pallas as pl
from jax.experimental.pallas import tpu as pltpu
from jax.experimental.pallas import tpu_sc as plsc

_RETENTION = 0.9
_SPEED = 5.0
_BS = 2048
_SC_CHUNK = 64


def _attn_body(x_ref, m_ref, out_ref, acc_ref, sumw_ref, nsteps):
    i = pl.program_id(0)
    m = m_ref[...]
    x = x_ref[...]

    num = lax.dot_general(x, m, (((1,), (1,)), ((), ())),
                          preferred_element_type=jnp.float32)
    x_norm = jnp.sqrt(jnp.sum(x * x, axis=1, keepdims=True))
    m_norm = jnp.sqrt(jnp.sum(m * m, axis=1)).reshape(1, -1)
    denom = jnp.maximum(x_norm * m_norm, 1e-8)
    w = jnp.exp(_SPEED * (num / denom))

    part = lax.dot_general(w, m, (((1,), (0,)), ((), ())),
                           preferred_element_type=jnp.float32)
    wsum = jnp.sum(w, axis=1, keepdims=True)

    @pl.when(i == 0)
    def _init():
        acc_ref[...] = part
        sumw_ref[...] = wsum

    @pl.when(i > 0)
    def _accum():
        acc_ref[...] += part
        sumw_ref[...] += wsum

    @pl.when(i == nsteps - 1)
    def _final():
        out_ref[...] = acc_ref[...] / sumw_ref[...]


def _attention(x, memory_bank):
    size, dim = memory_bank.shape
    b = x.shape[0]
    bs = _BS if size % _BS == 0 else size
    nsteps = size // bs
    return pl.pallas_call(
        functools.partial(_attn_body, nsteps=nsteps),
        grid=(nsteps,),
        in_specs=[
            pl.BlockSpec((b, dim), lambda i: (0, 0)),
            pl.BlockSpec((bs, dim), lambda i: (i, 0)),
        ],
        out_specs=pl.BlockSpec((b, dim), lambda i: (0, 0)),
        out_shape=jax.ShapeDtypeStruct((b, dim), jnp.float32),
        scratch_shapes=[
            pltpu.VMEM((b, dim), jnp.float32),
            pltpu.VMEM((b, 1), jnp.float32),
        ],
    )(x, memory_bank)


def _sc_update(x_new, importance, memory_bank):
    size, dim = memory_bank.shape
    info = plsc.get_sparse_core_info()
    ns = info.num_subcores
    nw = info.num_cores * ns
    rows_per_w = size // nw
    chunk = _SC_CHUNK
    nchunks = rows_per_w // chunk
    thresh = 1.0 - _RETENTION
    mesh = plsc.VectorSubcoreMesh(core_axis_name="c", subcore_axis_name="s")

    @functools.partial(
        pl.kernel, mesh=mesh,
        out_type=jax.ShapeDtypeStruct((size, dim), jnp.float32),
        scratch_types=[
            pltpu.VMEM((rows_per_w,), jnp.float32),
            pltpu.VMEM((chunk, dim), jnp.float32),
            pltpu.MemorySpace.VMEM_SHARED((ns, chunk, dim), jnp.float32),
            pltpu.SemaphoreType.DMA,
            pltpu.SemaphoreType.DMA,
            pltpu.SemaphoreType.DMA,
            pltpu.SemaphoreType.DMA,
        ],
    )
    def upd(xnew_hbm, imp_hbm, bank_hbm, out_hbm, imp_v, buf_a, spm,
            in_a, in_b, out_a, out_b):
        sid = lax.axis_index("s")
        wid = sid * info.num_cores + lax.axis_index("c")
        w_base = wid * rows_per_w
        pltpu.sync_copy(imp_hbm.at[pl.ds(w_base, rows_per_w)], imp_v)
        buf_b = spm.at[sid]

        def fill(c, buf, sem):
            base = w_base + c * chunk
            for g in range(chunk // 16):
                impv = imp_v[pl.ds(c * chunk + g * 16, 16)]
                for k in range(16):
                    r = g * 16 + k
                    s = impv[k]

                    @pl.when(s > thresh)
                    def _from_new():
                        pltpu.make_async_copy(
                            xnew_hbm.at[base + r], buf.at[r], sem).start()

                    @pl.when(jnp.logical_not(s > thresh))
                    def _from_bank():
                        pltpu.make_async_copy(
                            bank_hbm.at[base + r], buf.at[r], sem).start()

        def drain(buf, sem):
            pltpu.make_async_copy(
                bank_hbm.at[pl.ds(0, chunk), :], buf, sem).wait()

        def flush(c, buf, sem):
            pltpu.make_async_copy(
                buf, out_hbm.at[pl.ds(w_base + c * chunk, chunk), :],
                sem).start()

        def wait_flush(buf, sem):
            pltpu.make_async_copy(
                bank_hbm.at[pl.ds(0, chunk), :], buf, sem).wait()

        def do_pair(i, _):
            c0 = 2 * i
            c1 = 2 * i + 1

            @pl.when(i > 0)
            def _reuse_a():
                wait_flush(buf_a, out_a)
            fill(c0, buf_a, in_a)

            @pl.when(i > 0)
            def _reuse_b():
                wait_flush(buf_b, out_b)
            fill(c1, buf_b, in_b)

            drain(buf_a, in_a)
            flush(c0, buf_a, out_a)
            drain(buf_b, in_b)
            flush(c1, buf_b, out_b)
            return 0

        lax.fori_loop(0, nchunks // 2, do_pair, 0)
        wait_flush(buf_a, out_a)
        wait_flush(buf_b, out_b)

    return upd(x_new, importance, memory_bank)


def kernel(x, x_new, importance, memory_bank):
    out = _attention(x, memory_bank)
    new_bank = _sc_update(x_new, importance, memory_bank)
    return out, new_bank


# speedup vs baseline: 1.2350x; 1.0140x over previous
import functools

import jax
import jax.numpy as jnp
from jax import lax
from jax.experimental import pallas as pl
from jax.experimental.pallas import tpu as pltpu
from jax.experimental.pallas import tpu_sc as plsc

_RETENTION = 0.9
_SPEED = 5.0
_BS = 2048
_SC_CHUNK = 32


def _attn_body(x_ref, m_ref, out_ref, acc_ref, sumw_ref, nsteps):
    i = pl.program_id(0)
    m = m_ref[...]
    x = x_ref[...]

    num = lax.dot_general(x, m, (((1,), (1,)), ((), ())),
                          preferred_element_type=jnp.float32)
    x_norm = jnp.sqrt(jnp.sum(x * x, axis=1, keepdims=True))
    m_norm = jnp.sqrt(jnp.sum(m * m, axis=1)).reshape(1, -1)
    denom = jnp.maximum(x_norm * m_norm, 1e-8)
    w = jnp.exp(_SPEED * (num / denom))

    part = lax.dot_general(w, m, (((1,), (0,)), ((), ())),
                           preferred_element_type=jnp.float32)
    wsum = jnp.sum(w, axis=1, keepdims=True)

    @pl.when(i == 0)
    def _init():
        acc_ref[...] = part
        sumw_ref[...] = wsum

    @pl.when(i > 0)
    def _accum():
        acc_ref[...] += part
        sumw_ref[...] += wsum

    @pl.when(i == nsteps - 1)
    def _final():
        out_ref[...] = acc_ref[...] / sumw_ref[...]


def _attention(x, memory_bank):
    size, dim = memory_bank.shape
    b = x.shape[0]
    bs = _BS if size % _BS == 0 else size
    nsteps = size // bs
    return pl.pallas_call(
        functools.partial(_attn_body, nsteps=nsteps),
        grid=(nsteps,),
        in_specs=[
            pl.BlockSpec((b, dim), lambda i: (0, 0)),
            pl.BlockSpec((bs, dim), lambda i: (i, 0)),
        ],
        out_specs=pl.BlockSpec((b, dim), lambda i: (0, 0)),
        out_shape=jax.ShapeDtypeStruct((b, dim), jnp.float32),
        scratch_shapes=[
            pltpu.VMEM((b, dim), jnp.float32),
            pltpu.VMEM((b, 1), jnp.float32),
        ],
    )(x, memory_bank)


def _sc_update(x_new, importance, memory_bank):
    size, dim = memory_bank.shape
    info = plsc.get_sparse_core_info()
    nw = info.num_cores * info.num_subcores
    rows_per_w = size // nw
    chunk = _SC_CHUNK
    nchunks = rows_per_w // chunk
    thresh = 1.0 - _RETENTION
    mesh = plsc.VectorSubcoreMesh(core_axis_name="c", subcore_axis_name="s")

    @functools.partial(
        pl.kernel, mesh=mesh,
        out_type=jax.ShapeDtypeStruct((size, dim), jnp.float32),
        scratch_types=[
            pltpu.VMEM((rows_per_w,), jnp.float32),
            pltpu.VMEM((chunk, dim), jnp.float32),
            pltpu.VMEM((chunk, dim), jnp.float32),
            pltpu.VMEM((chunk, dim), jnp.float32),
            pltpu.VMEM((chunk, dim), jnp.float32),
            pltpu.SemaphoreType.DMA,
            pltpu.SemaphoreType.DMA,
            pltpu.SemaphoreType.DMA,
            pltpu.SemaphoreType.DMA,
            pltpu.SemaphoreType.DMA,
            pltpu.SemaphoreType.DMA,
            pltpu.SemaphoreType.DMA,
            pltpu.SemaphoreType.DMA,
        ],
    )
    def upd(xnew_hbm, imp_hbm, bank_hbm, out_hbm, imp_v, buf_a, buf_b,
            buf_c, buf_d, in_a, in_b, in_c, in_d,
            out_a, out_b, out_c, out_d):
        wid = lax.axis_index("s") * info.num_cores + lax.axis_index("c")
        w_base = wid * rows_per_w
        pltpu.sync_copy(imp_hbm.at[pl.ds(w_base, rows_per_w)], imp_v)

        def fill(c, buf, sem):
            base = w_base + c * chunk
            for g in range(chunk // 16):
                impv = imp_v[pl.ds(c * chunk + g * 16, 16)]
                for k in range(16):
                    r = g * 16 + k
                    s = impv[k]

                    @pl.when(s > thresh)
                    def _from_new():
                        pltpu.make_async_copy(
                            xnew_hbm.at[base + r], buf.at[r], sem).start()

                    @pl.when(jnp.logical_not(s > thresh))
                    def _from_bank():
                        pltpu.make_async_copy(
                            bank_hbm.at[base + r], buf.at[r], sem).start()

        def drain(buf, sem):
            pltpu.make_async_copy(
                bank_hbm.at[pl.ds(0, chunk), :], buf, sem).wait()

        def flush(c, buf, sem):
            pltpu.make_async_copy(
                buf, out_hbm.at[pl.ds(w_base + c * chunk, chunk), :],
                sem).start()

        def wait_flush(buf, sem):
            pltpu.make_async_copy(
                bank_hbm.at[pl.ds(0, chunk), :], buf, sem).wait()

        bufs = (buf_a, buf_b, buf_c, buf_d)
        in_sems = (in_a, in_b, in_c, in_d)
        out_sems = (out_a, out_b, out_c, out_d)

        def do_quad(i, _):
            for j in range(4):
                c = 4 * i + j

                @pl.when(i > 0)
                def _reuse():
                    wait_flush(bufs[j], out_sems[j])
                fill(c, bufs[j], in_sems[j])
            for j in range(4):
                drain(bufs[j], in_sems[j])
                flush(4 * i + j, bufs[j], out_sems[j])
            return 0

        lax.fori_loop(0, nchunks // 4, do_quad, 0)
        for j in range(4):
            wait_flush(bufs[j], out_sems[j])

    return upd(x_new, importance, memory_bank)


def kernel(x, x_new, importance, memory_bank):
    out = _attention(x, memory_bank)
    new_bank = _sc_update(x_new, importance, memory_bank)
    return out, new_bank
